# baseline (speedup 1.0000x reference)
import functools

import jax
import jax.numpy as jnp
from jax import lax
from jax.experimental import pallas as pl
from jax.experimental.pallas import tpu as pltpu
from jax.experimental.pallas import tpu_sc as plsc

N = 10000
E = 320000
D = 128
HG = 70
HGP = 128
HU = 80
HC = 128
C = 16
LAMB1 = 0.1
LAMB2 = 0.01

NC = 2
NS = 16
NW = NC * NS
K = 128
NR = 2560
EP = NR * K
RPT = NR // NW
RPS = NR // NS
BLK = 8
NP_DEN = 10240
NA = NP_DEN // NS
PT = 24

RE_C0 = 80
RE_C1 = 80

_mesh = functools.partial(
    plsc.VectorSubcoreMesh, core_axis_name="c", subcore_axis_name="s",
    num_cores=NC, num_subcores=NS)

_f32 = jnp.float32
_i32 = jnp.int32
_sc_tiled = pltpu.CompilerParams(
    needs_layout_passes=False, use_tc_tiling_on_sc=True)
_sc_linear = pltpu.CompilerParams(
    needs_layout_passes=False, use_tc_tiling_on_sc=False)


def _iota16():
  return lax.iota(_i32, 16)


def _tc1_body(x_ref, wg_ref, w1_ref, b1_ref, h_ref, y1_ref):
  xb = x_ref[...]
  h_ref[...] = jnp.dot(xb, wg_ref[...], preferred_element_type=_f32)
  y1_ref[...] = jnp.dot(xb, w1_ref[...], preferred_element_type=_f32) + b1_ref[...]


def _tc1(x, wgp, w1, b1):
  R = 1000
  return pl.pallas_call(
      _tc1_body,
      grid=(N // R,),
      in_specs=[
          pl.BlockSpec((R, D), lambda b: (b, 0)),
          pl.BlockSpec((D, HGP), lambda b: (0, 0)),
          pl.BlockSpec((D, HC), lambda b: (0, 0)),
          pl.BlockSpec((1, HC), lambda b: (0, 0)),
      ],
      out_specs=[
          pl.BlockSpec((R, HGP), lambda b: (b, 0)),
          pl.BlockSpec((R, HC), lambda b: (b, 0)),
      ],
      out_shape=[
          jax.ShapeDtypeStruct((N, HGP), _f32),
          jax.ShapeDtypeStruct((N, HC), _f32),
      ],
  )(x, wgp, w1, b1)


def _sc_edge_body(h_hbm, src_hbm, dst_hbm, a_hbm, e_out, sq_out,
                  av, sidx, didx, hs0, hs1, hd0, hd1,
                  ebuf, sqbuf, pte, pts, shs0, shs1, shd0, shd1):
  c = lax.axis_index("c")
  s = lax.axis_index("s")
  rowbase = jnp.where(c == 0, s * RE_C0, NS * RE_C0 + s * RE_C1)
  nblocks = jnp.where(c == 0, RE_C0 // BLK, RE_C1 // BLK)
  pltpu.sync_copy(a_hbm, av)
  av5 = tuple(av[pl.ds(16 * k, 16)] for k in range(HU // 16))
  hsb = (hs0, hs1)
  hdb = (hd0, hd1)
  shs = (shs0, shs1)
  shd = (shd0, shd1)
  lane15 = _iota16() * PT + 15

  def compute(hsr, hdr, j):
    def gbody(g, carry):
      for l in range(16):
        r = g * 16 + l
        pe = None
        ps = None
        for k in range(HU // 16):
          d = hsr[r, pl.dslice(16 * k, 16)] - hdr[r, pl.dslice(16 * k, 16)]
          t = jnp.abs(d) * av5[k]
          u = d * d
          pe = t if pe is None else pe + t
          ps = u if ps is None else ps + u
        pte[pl.dslice(l * PT, 16)] = plsc.cumsum(pe)
        pts[pl.dslice(l * PT, 16)] = plsc.cumsum(ps)
      ev = jnp.maximum(plsc.load_gather(pte, [lane15]), 0.0)
      sv = plsc.load_gather(pts, [lane15])
      ebuf[pl.dslice(j * K + g * 16, 16)] = ev
      sqbuf[pl.dslice(j * K + g * 16, 16)] = sv
      return carry

    lax.fori_loop(0, K // 16, gbody, 0)

  def block(bi, carry):
    row0 = rowbase + bi * BLK
    pltpu.sync_copy(src_hbm.at[pl.ds(row0, BLK)], sidx)
    pltpu.sync_copy(dst_hbm.at[pl.ds(row0, BLK)], didx)
    pltpu.async_copy(h_hbm.at[sidx.at[0]], hs0, shs0)
    pltpu.async_copy(h_hbm.at[didx.at[0]], hd0, shd0)

    def pair(p, carry2):
      for b in range(2):
        j = 2 * p + b
        nb = 1 - b

        @pl.when(j + 1 < BLK)
        def _issue():
          pltpu.async_copy(h_hbm.at[sidx.at[j + 1]], hsb[nb], shs[nb])
          pltpu.async_copy(h_hbm.at[didx.at[j + 1]], hdb[nb], shd[nb])

        pltpu.make_async_copy(h_hbm.at[sidx.at[j]], hsb[b], shs[b]).wait()
        pltpu.make_async_copy(h_hbm.at[didx.at[j]], hdb[b], shd[b]).wait()
        compute(hsb[b], hdb[b], j)
      return carry2

    lax.fori_loop(0, BLK // 2, pair, 0)
    pltpu.sync_copy(ebuf, e_out.at[pl.ds(row0 * K, BLK * K)])
    pltpu.sync_copy(sqbuf, sq_out.at[pl.ds(row0 * K, BLK * K)])
    return carry

  lax.fori_loop(0, nblocks, block, 0)


def _sc_edge(h, src2, dstg2, a_pad):
  return pl.kernel(
      _sc_edge_body,
      out_type=[
          jax.ShapeDtypeStruct((EP,), _f32),
          jax.ShapeDtypeStruct((EP,), _f32),
      ],
      mesh=_mesh(),
      scratch_types=[
          pltpu.VMEM((HGP,), _f32),
          pltpu.VMEM((BLK, K), _i32),
          pltpu.VMEM((BLK, K), _i32),
          pltpu.VMEM((K, HGP), _f32),
          pltpu.VMEM((K, HGP), _f32),
          pltpu.VMEM((K, HGP), _f32),
          pltpu.VMEM((K, HGP), _f32),
          pltpu.VMEM((BLK * K,), _f32),
          pltpu.VMEM((BLK * K,), _f32),
          pltpu.VMEM((16 * PT,), _f32),
          pltpu.VMEM((16 * PT,), _f32),
          pltpu.SemaphoreType.DMA,
          pltpu.SemaphoreType.DMA,
          pltpu.SemaphoreType.DMA,
          pltpu.SemaphoreType.DMA,
      ],
      compiler_params=_sc_tiled,
      name="sc_edge_scores",
  )(h, src2, dstg2, a_pad)


def _sc_soft_body(e_hbm, sq_hbm, dst_hbm, att_out, lp_out,
                  den_sh, max_sh, den_v, ebuf, didx, exbuf, sqbuf, attbuf,
                  zbuf, mbuf, lossbuf, sem_sc):
  c = lax.axis_index("c")
  s = lax.axis_index("s")
  wid = c * NS + s
  srow = s * RPS

  def amax(bi, m):
    pltpu.sync_copy(e_hbm.at[pl.ds((srow + bi * BLK) * K, BLK * K)], ebuf)
    for g in range(BLK * K // 16):
      m = jnp.maximum(m, ebuf[pl.dslice(g * 16, 16)])
    return m

  mvec = lax.fori_loop(0, RPS // BLK, amax, jnp.zeros((16,), _f32))
  mbuf[...] = mvec
  pltpu.sync_copy(mbuf, max_sh.at[s])

  for g in range(8):
    zbuf[pl.ds(g * 16, 16)] = jnp.zeros((16,), _f32)
  for j in range(NA // K):
    pltpu.sync_copy(zbuf, den_sh.at[pl.ds(s * NA + j * K, K)])
  plsc.subcore_barrier()

  def rmax(i, m):
    pltpu.sync_copy(max_sh.at[i], mbuf)
    return jnp.maximum(m, mbuf[...])

  mvec = lax.fori_loop(0, NS, rmax, jnp.zeros((16,), _f32))
  gmax = lax.reduce_max(mvec, (0,))

  def bstep(bi, carry):
    row0 = srow + bi * BLK
    pltpu.sync_copy(e_hbm.at[pl.ds(row0 * K, BLK * K)], ebuf)
    pltpu.sync_copy(dst_hbm.at[pl.ds(row0, BLK)], didx)

    def jexp(j, carry2):
      for g in range(K // 16):
        exbuf[j, pl.dslice(g * 16, 16)] = jnp.exp(
            ebuf[pl.dslice(j * K + g * 16, 16)] - gmax)
      return carry2

    lax.fori_loop(0, BLK, jexp, 0)

    def fire(j, carry2):
      pltpu.async_copy(exbuf.at[j], den_sh.at[didx.at[j]], sem_sc, add=True)
      return carry2

    lax.fori_loop(0, BLK, fire, 0)

    def drain(j, carry2):
      pltpu.make_async_copy(exbuf.at[j], den_sh.at[didx.at[j]], sem_sc).wait()
      return carry2

    lax.fori_loop(0, BLK, drain, 0)
    return carry

  lax.fori_loop(0, RPS // BLK, bstep, 0)
  plsc.subcore_barrier()

  pltpu.sync_copy(den_sh, den_v)
  rowbase = wid * RPT

  def cstep(bi, carry):
    l1, l2 = carry
    row0 = rowbase + bi * BLK
    pltpu.sync_copy(e_hbm.at[pl.ds(row0 * K, BLK * K)], ebuf)
    pltpu.sync_copy(dst_hbm.at[pl.ds(row0, BLK)], didx)
    pltpu.sync_copy(sq_hbm.at[pl.ds(row0 * K, BLK * K)], sqbuf)

    def jstep(j, carry2):
      l1, l2 = carry2
      for g in range(K // 16):
        sl = pl.dslice(j * K + g * 16, 16)
        ex = jnp.exp(ebuf[sl] - gmax)
        idx = didx[j, pl.dslice(g * 16, 16)]
        den = plsc.load_gather(den_v, [idx])
        at = ex / (den + 1e-16)
        attbuf[sl] = at
        gidx = (row0 + j) * K + g * 16 + _iota16()
        msk = gidx < E
        l1 = l1 + jnp.where(msk, at * sqbuf[sl], 0.0)
        l2 = l2 + jnp.where(msk, at * at, 0.0)
      return l1, l2

    l1, l2 = lax.fori_loop(0, BLK, jstep, (l1, l2))
    pltpu.sync_copy(attbuf, att_out.at[pl.ds(row0 * K, BLK * K)])
    return l1, l2

  z16 = jnp.zeros((16,), _f32)
  l1, l2 = lax.fori_loop(0, RPT // BLK, cstep, (z16, z16))
  lossbuf[pl.ds(0, 16)] = l1
  lossbuf[pl.ds(16, 16)] = l2
  pltpu.sync_copy(lossbuf, lp_out.at[pl.ds(wid * 32, 32)])


def _sc_soft(e1, sq1, dsts2):
  return pl.kernel(
      _sc_soft_body,
      out_type=[
          jax.ShapeDtypeStruct((EP,), _f32),
          jax.ShapeDtypeStruct((NW * 32,), _f32),
      ],
      mesh=_mesh(),
      scratch_types=[
          pltpu.VMEM_SHARED((NP_DEN,), _f32),
          pltpu.VMEM_SHARED((NS, 16), _f32),
          pltpu.VMEM((NP_DEN,), _f32),
          pltpu.VMEM((BLK * K,), _f32),
          pltpu.VMEM((BLK, K), _i32),
          pltpu.VMEM((BLK, K), _f32),
          pltpu.VMEM((BLK * K,), _f32),
          pltpu.VMEM((BLK * K,), _f32),
          pltpu.VMEM((K,), _f32),
          pltpu.VMEM((16,), _f32),
          pltpu.VMEM((32,), _f32),
          pltpu.SemaphoreType.DMA,
      ],
      compiler_params=_sc_tiled,
      name="sc_segment_softmax",
  )(e1, sq1, dsts2)


def _scale_rows(rref, attb, j, F):
  def gbody(g, carry):
    atv = attb[pl.dslice(j * K + g * 16, 16)]
    for l in range(16):
      asp = jnp.full((16,), atv[l], _f32)
      r = g * 16 + l
      for k in range(F // 16):
        sl = pl.dslice(k * 16, 16)
        rref[r, sl] = rref[r, sl] * asp
    return carry

  lax.fori_loop(0, K // 16, gbody, 0)


def _spmm_epilogue(acc_sh, out_hbm, c, s):
  for j in range(NA // K):
    start = s * NA + j * K
    pltpu.sync_copy(acc_sh.at[pl.ds(start, K)],
                    out_hbm.at[pl.ds(c * NP_DEN + start, K)])


def _spmm_zero(acc_sh, rows0, s, F):
  def zrow(r, carry):
    for k in range(F // 16):
      rows0[r, pl.dslice(k * 16, 16)] = jnp.zeros((16,), _f32)
    return carry

  lax.fori_loop(0, K, zrow, 0)
  for j in range(NA // K):
    pltpu.sync_copy(rows0, acc_sh.at[pl.ds(s * NA + j * K, K)])
  plsc.subcore_barrier()


def _sc_spmm1_body(RS0, RS1, src_hbm, dst_hbm, att_hbm, y_hbm, out_hbm,
                   acc_sh, sidx, didx, attb, rows0, rows1, sg0, sg1):
  c = lax.axis_index("c")
  s = lax.axis_index("s")
  rowsb = (rows0, rows1)
  sg = (sg0, sg1)
  _spmm_zero(acc_sh, rows0, s, HC)
  rowbase = jnp.where(c == 0, s * RS0, NS * RS0 + s * RS1)
  nblocks = jnp.where(c == 0, RS0 // BLK, RS1 // BLK)

  def block(bi, carry):
    row0 = rowbase + bi * BLK
    pltpu.sync_copy(src_hbm.at[pl.ds(row0, BLK)], sidx)
    pltpu.sync_copy(dst_hbm.at[pl.ds(row0, BLK)], didx)
    pltpu.sync_copy(att_hbm.at[pl.ds(row0 * K, BLK * K)],
                    attb.at[pl.ds(0, BLK * K)])
    pltpu.async_copy(y_hbm.at[sidx.at[0]], rows0, sg0)

    def pair(p, carry2):
      for b in range(2):
        j = 2 * p + b
        nb = 1 - b

        @pl.when(j + 1 < BLK)
        def _issue():
          pltpu.async_copy(y_hbm.at[sidx.at[j + 1]], rowsb[nb], sg[nb])

        pltpu.make_async_copy(y_hbm.at[sidx.at[j]], rowsb[b], sg[b]).wait()
        _scale_rows(rowsb[b], attb, j, HC)
        pltpu.sync_copy(rowsb[b], acc_sh.at[didx.at[j]], add=True)
      return carry2

    lax.fori_loop(0, BLK // 2, pair, 0)
    return carry

  lax.fori_loop(0, nblocks, block, 0)
  plsc.subcore_barrier()
  _spmm_epilogue(acc_sh, out_hbm, c, s)


def _sc_spmm1(RS0, RS1, src2, dsts2, att1, y):
  return pl.kernel(
      functools.partial(_sc_spmm1_body, RS0, RS1),
      out_type=jax.ShapeDtypeStruct((NC * NP_DEN, HC), _f32),
      mesh=_mesh(),
      scratch_types=[
          pltpu.VMEM_SHARED((NP_DEN, HC), _f32),
          pltpu.VMEM((BLK, K), _i32),
          pltpu.VMEM((BLK, K), _i32),
          pltpu.VMEM((BLK * K + 16,), _f32),
          pltpu.VMEM((K, HC), _f32),
          pltpu.VMEM((K, HC), _f32),
          pltpu.SemaphoreType.DMA,
          pltpu.SemaphoreType.DMA,
      ],
      compiler_params=_sc_tiled,
      name="sc_spmm_128",
  )(src2, dsts2, att1, y)


def _sc_spmm2_body(RS0, RS1, src_hbm, dst_hbm, att_hbm, y_hbm, out_hbm,
                   acc_sh, sidx, didx, attb, rows0, rows1, sg0, sg1):
  c = lax.axis_index("c")
  s = lax.axis_index("s")
  rowsb = (rows0, rows1)
  sg = (sg0, sg1)
  _spmm_zero(acc_sh, rows0, s, C)
  rowbase = jnp.where(c == 0, s * RS0, NS * RS0 + s * RS1)
  nblocks = jnp.where(c == 0, RS0 // BLK, RS1 // BLK)

  def block(bi, carry):
    row0 = rowbase + bi * BLK
    pltpu.sync_copy(src_hbm.at[pl.ds(row0 * K, BLK * K)], sidx)
    pltpu.sync_copy(dst_hbm.at[pl.ds(row0, BLK)], didx)
    pltpu.sync_copy(att_hbm.at[pl.ds(row0 * K, BLK * K)],
                    attb.at[pl.ds(0, BLK * K)])
    pltpu.async_copy(y_hbm.at[sidx.at[pl.ds(0, K)]], rows0, sg0)

    def pair(p, carry2):
      for b in range(2):
        j = 2 * p + b
        nb = 1 - b

        @pl.when(j + 1 < BLK)
        def _issue():
          pltpu.async_copy(
              y_hbm.at[sidx.at[pl.ds((j + 1) * K, K)]], rowsb[nb], sg[nb])

        pltpu.make_async_copy(
            y_hbm.at[sidx.at[pl.ds(j * K, K)]], rowsb[b], sg[b]).wait()
        _scale_rows(rowsb[b], attb, j, C)
        pltpu.sync_copy(rowsb[b], acc_sh.at[didx.at[j]], add=True)
      return carry2

    lax.fori_loop(0, BLK // 2, pair, 0)
    return carry

  lax.fori_loop(0, nblocks, block, 0)
  plsc.subcore_barrier()
  _spmm_epilogue(acc_sh, out_hbm, c, s)


def _sc_spmm2(RS0, RS1, src1, db2, att1, y):
  return pl.kernel(
      functools.partial(_sc_spmm2_body, RS0, RS1),
      out_type=jax.ShapeDtypeStruct((NC * NP_DEN, C), _f32),
      mesh=_mesh(),
      scratch_types=[
          pltpu.VMEM_SHARED((NP_DEN, C), _f32),
          pltpu.VMEM((BLK * K,), _i32),
          pltpu.VMEM((BLK, K), _i32),
          pltpu.VMEM((BLK * K + 16,), _f32),
          pltpu.VMEM((K, C), _f32),
          pltpu.VMEM((K, C), _f32),
          pltpu.SemaphoreType.DMA,
          pltpu.SemaphoreType.DMA,
      ],
      compiler_params=_sc_linear,
      name="sc_spmm_16",
  )(src1, db2, att1, y)


def _tc2_body(acc_ref, w2_ref, b2_ref, y2_ref):
  z1 = jnp.maximum(acc_ref[0] + acc_ref[1], 0.0)
  y2_ref[...] = jnp.dot(z1, w2_ref[...], preferred_element_type=_f32) + b2_ref[...]


def _tc2(acc, w2, b2):
  R = 1000
  return pl.pallas_call(
      _tc2_body,
      grid=(N // R,),
      in_specs=[
          pl.BlockSpec((2, R, HC), lambda b: (0, b, 0)),
          pl.BlockSpec((HC, C), lambda b: (0, 0)),
          pl.BlockSpec((1, C), lambda b: (0, 0)),
      ],
      out_specs=pl.BlockSpec((R, C), lambda b: (b, 0)),
      out_shape=jax.ShapeDtypeStruct((N, C), _f32),
  )(acc, w2, b2)


def _tc3_body(acc_ref, lp_ref, z_ref, loss_ref):
  z_ref[...] = acc_ref[0] + acc_ref[1]

  @pl.when(pl.program_id(0) == 0)
  def _():
    lp = lp_ref[...]
    l1 = jnp.sum(lp[:, :16])
    l2 = jnp.sum(lp[:, 16:])
    loss_ref[...] = jnp.reshape(
        (LAMB1 * l1 + LAMB2 * l2) / float(N * N), (1, 1))


def _tc3(acc, lp):
  R = 1000
  return pl.pallas_call(
      _tc3_body,
      grid=(N // R,),
      in_specs=[
          pl.BlockSpec((2, R, C), lambda b: (0, b, 0)),
          pl.BlockSpec((NW, 32), lambda b: (0, 0)),
      ],
      out_specs=[
          pl.BlockSpec((R, C), lambda b: (b, 0)),
          pl.BlockSpec((1, 1), lambda b: (0, 0)),
      ],
      out_shape=[
          jax.ShapeDtypeStruct((N, C), _f32),
          jax.ShapeDtypeStruct((1, 1), _f32),
      ],
  )(acc, lp)


def kernel(x, edge_index, Wg, a, W1, b1, W2, b2):
  src = edge_index[0]
  dst = edge_index[1]
  pad = EP - E
  zpad = jnp.zeros((pad,), _i32)
  src1 = jnp.concatenate([src, zpad])
  dstg1 = jnp.concatenate([dst, zpad])
  dsts1 = jnp.concatenate(
      [dst, N + (jnp.arange(pad, dtype=_i32) % (NP_DEN - N))])
  src2 = src1.reshape(NR, K)
  dstg2 = dstg1.reshape(NR, K)
  dsts2 = dsts1.reshape(NR, K)
  db2 = lax.optimization_barrier(dsts1).reshape(NR, K)
  wgp = jnp.pad(Wg, ((0, 0), (0, HGP - HG)))
  a_pad = jnp.pad(a[:, 0], (0, HGP - HG))

  h, y1 = _tc1(x, wgp, W1, b1.reshape(1, HC))
  e1, sq1 = _sc_edge(h, src2, dstg2, a_pad)
  att1, lp1 = _sc_soft(e1, sq1, dsts2)
  acc1 = _sc_spmm1(80, 80, src2, dsts2, att1, y1)
  y2 = _tc2(acc1.reshape(NC, NP_DEN, HC), W2, b2.reshape(1, C))
  acc2 = _sc_spmm2(80, 80, src1, db2, att1, y2)
  z, loss = _tc3(acc2.reshape(NC, NP_DEN, C), lp1.reshape(NW, 32))
  att = att1[:E]
  return z, att, loss[0, 0]


# speedup vs baseline: 1.2881x; 1.2881x over previous
import functools

import jax
import jax.numpy as jnp
from jax import lax
from jax.experimental import pallas as pl
from jax.experimental.pallas import tpu as pltpu
from jax.experimental.pallas import tpu_sc as plsc

N = 10000
E = 320000
D = 128
HG = 70
HGP = 80
HC = 128
C = 16
LAMB1 = 0.1
LAMB2 = 0.01

NC = 2
NS = 16
NW = NC * NS
K = 128
NR = 2560
EP = NR * K
RPT = NR // NW
RPS = NR // NS
BLK = 10
NBG = RPT // BLK
NBS = RPS // BLK
NP_DEN = 10240
NA = NP_DEN // NS

_mesh = functools.partial(
    plsc.VectorSubcoreMesh, core_axis_name="c", subcore_axis_name="s",
    num_cores=NC, num_subcores=NS)

_f32 = jnp.float32
_i32 = jnp.int32
_sc_params = pltpu.CompilerParams(
    needs_layout_passes=False, use_tc_tiling_on_sc=False)


def _iota16():
  return lax.iota(_i32, 16)


def _tc1_body(x_ref, wg_ref, w1_ref, b1_ref, h_ref, y1_ref):
  xb = x_ref[...]
  h_ref[...] = jnp.dot(xb, wg_ref[...], preferred_element_type=_f32)
  y1_ref[...] = jnp.dot(xb, w1_ref[...], preferred_element_type=_f32) + b1_ref[...]


def _tc1(x, wgp, w1, b1):
  R = 1000
  return pl.pallas_call(
      _tc1_body,
      grid=(N // R,),
      in_specs=[
          pl.BlockSpec((R, D), lambda b: (b, 0)),
          pl.BlockSpec((D, HGP), lambda b: (0, 0)),
          pl.BlockSpec((D, HC), lambda b: (0, 0)),
          pl.BlockSpec((1, HC), lambda b: (0, 0)),
      ],
      out_specs=[
          pl.BlockSpec((R, HGP), lambda b: (b, 0)),
          pl.BlockSpec((R, HC), lambda b: (b, 0)),
      ],
      out_shape=[
          jax.ShapeDtypeStruct((N, HGP), _f32),
          jax.ShapeDtypeStruct((N, HC), _f32),
      ],
  )(x, wgp, w1, b1)


EBLK = 8
EDEPTH = 4
RE_C0 = 136
RE_C1 = 24
PT = 24


def _sc_edge_body(h_hbm, src_hbm, dst_hbm, a_hbm, e_out, sq_out, mx_out,
                  av, sidx, didx, hs0, hs1, hs2, hs3, hd0, hd1, hd2, hd3,
                  ebuf, sqbuf, mbuf, pte, pts, shs0, shs1, shs2, shs3,
                  shd0, shd1, shd2, shd3):
  c = lax.axis_index("c")
  s = lax.axis_index("s")
  wid = c * NS + s
  rowbase = jnp.where(c == 0, s * RE_C0, NS * RE_C0 + s * RE_C1)
  nblocks = jnp.where(c == 0, RE_C0 // EBLK, RE_C1 // EBLK)
  pltpu.sync_copy(a_hbm, av)
  av5 = tuple(av[pl.ds(16 * k, 16)] for k in range(HGP // 16))
  hsb = (hs0, hs1, hs2, hs3)
  hdb = (hd0, hd1, hd2, hd3)
  shs = (shs0, shs1, shs2, shs3)
  shd = (shd0, shd1, shd2, shd3)
  lane15 = _iota16() * PT + 15

  def compute(hsr, hdr, j, m):
    def gbody(g, m):
      for l in range(16):
        r = g * 16 + l
        hs_k = [hsr[r, pl.dslice(16 * k, 16)] for k in range(HGP // 16)]
        hd_k = [hdr[r, pl.dslice(16 * k, 16)] for k in range(HGP // 16)]
        pe = None
        ps = None
        for k in range(HGP // 16):
          d = hs_k[k] - hd_k[k]
          t = jnp.abs(d) * av5[k]
          u = d * d
          pe = t if pe is None else pe + t
          ps = u if ps is None else ps + u
        pte[pl.dslice(l * PT, 16)] = plsc.cumsum(pe)
        pts[pl.dslice(l * PT, 16)] = plsc.cumsum(ps)
      ev = jnp.maximum(plsc.load_gather(pte, [lane15]), 0.0)
      sv = plsc.load_gather(pts, [lane15])
      ebuf[j, pl.dslice(g * 16, 16)] = ev
      sqbuf[j, pl.dslice(g * 16, 16)] = sv
      return jnp.maximum(m, ev)

    return lax.fori_loop(0, K // 16, gbody, m)

  def block(bi, m):
    row0 = rowbase + bi * EBLK
    pltpu.sync_copy(src_hbm.at[pl.ds(row0, EBLK)], sidx)
    pltpu.sync_copy(dst_hbm.at[pl.ds(row0, EBLK)], didx)
    for j in range(EDEPTH - 1):
      pltpu.async_copy(h_hbm.at[sidx.at[j]], hsb[j], shs[j])
      pltpu.async_copy(h_hbm.at[didx.at[j]], hdb[j], shd[j])

    def quad(p, m):
      for b in range(EDEPTH):
        j = EDEPTH * p + b
        ib = (b + EDEPTH - 1) % EDEPTH

        @pl.when(j + EDEPTH - 1 < EBLK)
        def _issue():
          pltpu.async_copy(h_hbm.at[sidx.at[j + EDEPTH - 1]], hsb[ib], shs[ib])
          pltpu.async_copy(h_hbm.at[didx.at[j + EDEPTH - 1]], hdb[ib], shd[ib])

        pltpu.make_async_copy(h_hbm.at[sidx.at[j]], hsb[b], shs[b]).wait()
        pltpu.make_async_copy(h_hbm.at[didx.at[j]], hdb[b], shd[b]).wait()
        m = compute(hsb[b], hdb[b], j, m)
      return m

    m = lax.fori_loop(0, EBLK // EDEPTH, quad, m)
    pltpu.sync_copy(ebuf, e_out.at[pl.ds(row0, EBLK)])
    pltpu.sync_copy(sqbuf, sq_out.at[pl.ds(row0, EBLK)])
    return m

  m = lax.fori_loop(0, nblocks, block, jnp.zeros((16,), _f32))
  mbuf[...] = m
  pltpu.sync_copy(mbuf, mx_out.at[wid])


def _sc_edge(h, src2, dstg2, a_pad):
  return pl.kernel(
      _sc_edge_body,
      out_type=[
          jax.ShapeDtypeStruct((NR, K), _f32),
          jax.ShapeDtypeStruct((NR, K), _f32),
          jax.ShapeDtypeStruct((NW, 16), _f32),
      ],
      mesh=_mesh(),
      scratch_types=[
          pltpu.VMEM((HGP,), _f32),
          pltpu.VMEM((EBLK, K), _i32),
          pltpu.VMEM((EBLK, K), _i32),
          pltpu.VMEM((K, HGP), _f32),
          pltpu.VMEM((K, HGP), _f32),
          pltpu.VMEM((K, HGP), _f32),
          pltpu.VMEM((K, HGP), _f32),
          pltpu.VMEM((K, HGP), _f32),
          pltpu.VMEM((K, HGP), _f32),
          pltpu.VMEM((K, HGP), _f32),
          pltpu.VMEM((K, HGP), _f32),
          pltpu.VMEM((EBLK, K), _f32),
          pltpu.VMEM((EBLK, K), _f32),
          pltpu.VMEM((16,), _f32),
          pltpu.VMEM((16 * PT,), _f32),
          pltpu.VMEM((16 * PT,), _f32),
          pltpu.SemaphoreType.DMA,
          pltpu.SemaphoreType.DMA,
          pltpu.SemaphoreType.DMA,
          pltpu.SemaphoreType.DMA,
          pltpu.SemaphoreType.DMA,
          pltpu.SemaphoreType.DMA,
          pltpu.SemaphoreType.DMA,
          pltpu.SemaphoreType.DMA,
      ],
      compiler_params=_sc_params,
      name="sc_edge_scores",
  )(h, src2, dstg2, a_pad)


def _sc_soft_body(e_hbm, sq_hbm, dst_hbm, mx_hbm, att_out, lp_out,
                  den_sh, den_v, mxv, ebuf, didx, exbuf, sqbuf, attbuf,
                  zbuf, lossbuf, sem_sc):
  c = lax.axis_index("c")
  s = lax.axis_index("s")
  wid = c * NS + s

  pltpu.sync_copy(mx_hbm, mxv)

  def rmax(i, m):
    return jnp.maximum(m, mxv[i, :])

  mvec = lax.fori_loop(0, NW, rmax, jnp.zeros((16,), _f32))
  gmax = lax.reduce_max(mvec, (0,))

  for g in range(8):
    zbuf[pl.ds(g * 16, 16)] = jnp.zeros((16,), _f32)
  for j in range(NA // K):
    pltpu.sync_copy(zbuf, den_sh.at[pl.ds(s * NA + j * K, K)])
  plsc.subcore_barrier()

  def bstep(bi, carry):
    row0 = s * RPS + bi * BLK
    pltpu.sync_copy(e_hbm.at[pl.ds(row0, BLK)], ebuf)
    pltpu.sync_copy(dst_hbm.at[pl.ds(row0, BLK)], didx)

    def jexp(j, carry2):
      for g in range(K // 16):
        exbuf[j, pl.dslice(g * 16, 16)] = jnp.exp(
            ebuf[j, pl.dslice(g * 16, 16)] - gmax)
      return carry2

    lax.fori_loop(0, BLK, jexp, 0)

    def fire(j, carry2):
      pltpu.async_copy(exbuf.at[j], den_sh.at[didx.at[j]], sem_sc, add=True)
      return carry2

    lax.fori_loop(0, BLK, fire, 0)

    def drain(j, carry2):
      pltpu.make_async_copy(exbuf.at[j], den_sh.at[didx.at[j]], sem_sc).wait()
      return carry2

    lax.fori_loop(0, BLK, drain, 0)
    return carry

  lax.fori_loop(0, NBS, bstep, 0)
  plsc.subcore_barrier()

  pltpu.sync_copy(den_sh, den_v)
  rowbase = wid * RPT

  def cstep(bi, carry):
    l1, l2 = carry
    row0 = rowbase + bi * BLK
    pltpu.sync_copy(e_hbm.at[pl.ds(row0, BLK)], ebuf)
    pltpu.sync_copy(dst_hbm.at[pl.ds(row0, BLK)], didx)
    pltpu.sync_copy(sq_hbm.at[pl.ds(row0, BLK)], sqbuf)

    def jstep(j, carry2):
      l1, l2 = carry2
      for g in range(K // 16):
        sl = pl.dslice(g * 16, 16)
        ex = jnp.exp(ebuf[j, sl] - gmax)
        idx = didx[j, sl]
        den = plsc.load_gather(den_v, [idx])
        at = ex / (den + 1e-16)
        attbuf[j, sl] = at
        gidx = (row0 + j) * K + g * 16 + _iota16()
        msk = gidx < E
        l1 = l1 + jnp.where(msk, at * sqbuf[j, sl], 0.0)
        l2 = l2 + jnp.where(msk, at * at, 0.0)
      return l1, l2

    l1, l2 = lax.fori_loop(0, BLK, jstep, (l1, l2))
    pltpu.sync_copy(attbuf, att_out.at[pl.ds(row0, BLK)])
    return l1, l2

  z16 = jnp.zeros((16,), _f32)
  l1, l2 = lax.fori_loop(0, NBG, cstep, (z16, z16))
  lossbuf[pl.ds(0, 16)] = l1
  lossbuf[pl.ds(16, 16)] = l2
  pltpu.sync_copy(lossbuf, lp_out.at[wid])


def _sc_soft(e2, sq2, dsts2, mx):
  return pl.kernel(
      _sc_soft_body,
      out_type=[
          jax.ShapeDtypeStruct((NR, K), _f32),
          jax.ShapeDtypeStruct((NW, 32), _f32),
      ],
      mesh=_mesh(),
      scratch_types=[
          pltpu.VMEM_SHARED((NP_DEN,), _f32),
          pltpu.VMEM((NP_DEN,), _f32),
          pltpu.VMEM((NW, 16), _f32),
          pltpu.VMEM((BLK, K), _f32),
          pltpu.VMEM((BLK, K), _i32),
          pltpu.VMEM((BLK, K), _f32),
          pltpu.VMEM((BLK, K), _f32),
          pltpu.VMEM((BLK, K), _f32),
          pltpu.VMEM((K,), _f32),
          pltpu.VMEM((32,), _f32),
          pltpu.SemaphoreType.DMA,
      ],
      compiler_params=_sc_params,
      name="sc_segment_softmax",
  )(e2, sq2, dsts2, mx)


def _sc_spmm_body(F, RS0, RS1, src_hbm, dst_hbm, att_hbm, y_hbm, out_hbm,
                  acc_sh, sidx, didx, attb, rows0, rows1,
                  sg0, sg1, ss0, ss1):
  c = lax.axis_index("c")
  s = lax.axis_index("s")
  rowsb = (rows0, rows1)
  sg = (sg0, sg1)
  del ss0, ss1

  def run():
    def zrow(r, carry):
      for k in range(F // 16):
        rows0[r, pl.dslice(k * 16, 16)] = jnp.zeros((16,), _f32)
      return carry

    lax.fori_loop(0, K, zrow, 0)
    for j in range(NA // K):
      pltpu.sync_copy(rows0, acc_sh.at[pl.ds(s * NA + j * K, K)])
    plsc.subcore_barrier()
    if RS1 == 0:
      rowbase = s * RS0
      nblocks = RS0 // BLK
    else:
      rowbase = jnp.where(c == 0, s * RS0, NS * RS0 + s * RS1)
      nblocks = jnp.where(c == 0, RS0 // BLK, RS1 // BLK)

    def scale(rref, j):
      def gbody(g, carry):
        atv = attb[j, pl.dslice(g * 16, 16)]
        for l in range(16):
          asp = jnp.full((16,), atv[l], _f32)
          r = g * 16 + l
          for k in range(F // 16):
            sl = pl.dslice(k * 16, 16)
            rref[r, sl] = rref[r, sl] * asp
        return carry

      lax.fori_loop(0, K // 16, gbody, 0)

    def block(bi, carry):
      row0 = rowbase + bi * BLK
      pltpu.sync_copy(src_hbm.at[pl.ds(row0, BLK)], sidx)
      pltpu.sync_copy(dst_hbm.at[pl.ds(row0, BLK)], didx)
      pltpu.sync_copy(att_hbm.at[pl.ds(row0, BLK)], attb)
      pltpu.async_copy(y_hbm.at[sidx.at[0]], rows0, sg0)

      def pair(p, carry2):
        for b in range(2):
          j = 2 * p + b
          nb = 1 - b

          @pl.when(j + 1 < BLK)
          def _issue():
            pltpu.async_copy(y_hbm.at[sidx.at[j + 1]], rowsb[nb], sg[nb])

          pltpu.make_async_copy(y_hbm.at[sidx.at[j]], rowsb[b], sg[b]).wait()
          scale(rowsb[b], j)
          pltpu.sync_copy(rowsb[b], acc_sh.at[didx.at[j]], add=True)
        return carry2

      lax.fori_loop(0, BLK // 2, pair, 0)
      return carry

    lax.fori_loop(0, nblocks, block, 0)
    plsc.subcore_barrier()

    for j in range(NA // K):
      start = s * NA + j * K
      if RS1 == 0:
        pltpu.sync_copy(acc_sh.at[pl.ds(start, K)],
                        out_hbm.at[pl.ds(start, K)])
      else:
        pltpu.sync_copy(acc_sh.at[pl.ds(start, K)],
                        out_hbm.at[pl.ds(c * NP_DEN + start, K)])

  if RS1 == 0:
    @pl.when(c == 0)
    def _run0():
      run()
  else:
    run()


def _sc_spmm(F, RS0, RS1, src2, dsts2, att2, y):
  nout = NP_DEN if RS1 == 0 else NC * NP_DEN
  return pl.kernel(
      functools.partial(_sc_spmm_body, F, RS0, RS1),
      out_type=jax.ShapeDtypeStruct((nout, F), _f32),
      mesh=_mesh(),
      scratch_types=[
          pltpu.VMEM_SHARED((NP_DEN, F), _f32),
          pltpu.VMEM((BLK, K), _i32),
          pltpu.VMEM((BLK, K), _i32),
          pltpu.VMEM((BLK, K), _f32),
          pltpu.VMEM((K, F), _f32),
          pltpu.VMEM((K, F), _f32),
          pltpu.SemaphoreType.DMA,
          pltpu.SemaphoreType.DMA,
          pltpu.SemaphoreType.DMA,
          pltpu.SemaphoreType.DMA,
      ],
      compiler_params=_sc_params,
      name=f"sc_spmm_{F}",
  )(src2, dsts2, att2, y)


def _tc2_body(acc_ref, w2_ref, b2_ref, y2_ref):
  z1 = jnp.maximum(acc_ref[...], 0.0)
  y2_ref[...] = jnp.dot(z1, w2_ref[...], preferred_element_type=_f32) + b2_ref[...]


def _tc2(acc, w2, b2):
  R = 1000
  return pl.pallas_call(
      _tc2_body,
      grid=(N // R,),
      in_specs=[
          pl.BlockSpec((R, HC), lambda b: (b, 0)),
          pl.BlockSpec((HC, C), lambda b: (0, 0)),
          pl.BlockSpec((1, C), lambda b: (0, 0)),
      ],
      out_specs=pl.BlockSpec((R, C), lambda b: (b, 0)),
      out_shape=jax.ShapeDtypeStruct((N, C), _f32),
  )(acc, w2, b2)


def _tc3_body(acc_ref, lp_ref, z_ref, loss_ref):
  z_ref[...] = acc_ref[0] + acc_ref[1]

  @pl.when(pl.program_id(0) == 0)
  def _():
    lp = lp_ref[...]
    l1 = jnp.sum(lp[:, :16])
    l2 = jnp.sum(lp[:, 16:])
    loss_ref[...] = jnp.reshape(
        (LAMB1 * l1 + LAMB2 * l2) / float(N * N), (1, 1))


def _tc3(acc, lp):
  R = 1000
  return pl.pallas_call(
      _tc3_body,
      grid=(N // R,),
      in_specs=[
          pl.BlockSpec((2, R, C), lambda b: (0, b, 0)),
          pl.BlockSpec((NW, 32), lambda b: (0, 0)),
      ],
      out_specs=[
          pl.BlockSpec((R, C), lambda b: (b, 0)),
          pl.BlockSpec((1, 1), lambda b: (0, 0)),
      ],
      out_shape=[
          jax.ShapeDtypeStruct((N, C), _f32),
          jax.ShapeDtypeStruct((1, 1), _f32),
      ],
  )(acc, lp)


def kernel(x, edge_index, Wg, a, W1, b1, W2, b2):
  src = edge_index[0]
  dst = edge_index[1]
  pad = EP - E
  zpad = jnp.zeros((pad,), _i32)
  src2 = jnp.concatenate([src, zpad]).reshape(NR, K)
  dstg2 = jnp.concatenate([dst, zpad]).reshape(NR, K)
  dsts2 = jnp.concatenate(
      [dst, N + (jnp.arange(pad, dtype=_i32) % (NP_DEN - N))]).reshape(NR, K)
  wgp = jnp.pad(Wg, ((0, 0), (0, HGP - HG)))
  a_pad = jnp.pad(a[:, 0], (0, HGP - HG))

  h, y1 = _tc1(x, wgp, W1, b1.reshape(1, HC))
  e2, sq2, mx = _sc_edge(h, src2, dstg2, a_pad)
  att2, lossparts = _sc_soft(e2, sq2, dsts2, mx)
  acc1 = _sc_spmm(HC, 160, 0, src2, dsts2, att2, y1)
  y2 = _tc2(acc1, W2, b2.reshape(1, C))
  acc2 = _sc_spmm(C, 90, 70, src2, dsts2, att2, y2)
  z, loss = _tc3(acc2.reshape(NC, NP_DEN, C), lossparts)
  att = att2.reshape(EP)[:E]
  return z, att, loss[0, 0]


# speedup vs baseline: 1.4561x; 1.1304x over previous
import functools

import jax
import jax.numpy as jnp
from jax import lax
from jax.experimental import pallas as pl
from jax.experimental.pallas import tpu as pltpu
from jax.experimental.pallas import tpu_sc as plsc

N = 10000
E = 320000
D = 128
HG = 70
HGP = 80
HC = 128
C = 16
LAMB1 = 0.1
LAMB2 = 0.01

NC = 2
NS = 16
NW = NC * NS
K = 128
NR = 2560
EP = NR * K
RPT = NR // NW
RPS = NR // NS
BLK = 10
NBG = RPT // BLK
NBS = RPS // BLK
NP_DEN = 10240
NA = NP_DEN // NS

_mesh = functools.partial(
    plsc.VectorSubcoreMesh, core_axis_name="c", subcore_axis_name="s",
    num_cores=NC, num_subcores=NS)

_f32 = jnp.float32
_i32 = jnp.int32
_sc_params = pltpu.CompilerParams(
    needs_layout_passes=False, use_tc_tiling_on_sc=False)


def _iota16():
  return lax.iota(_i32, 16)


def _tc1_body(x_ref, wg_ref, w1_ref, b1_ref, h_ref, y1_ref):
  xb = x_ref[...]
  h_ref[...] = jnp.dot(xb, wg_ref[...], preferred_element_type=_f32)
  y1_ref[...] = jnp.dot(xb, w1_ref[...], preferred_element_type=_f32) + b1_ref[...]


def _tc1(x, wgp, w1, b1):
  R = 1000
  return pl.pallas_call(
      _tc1_body,
      grid=(N // R,),
      in_specs=[
          pl.BlockSpec((R, D), lambda b: (b, 0)),
          pl.BlockSpec((D, HGP), lambda b: (0, 0)),
          pl.BlockSpec((D, HC), lambda b: (0, 0)),
          pl.BlockSpec((1, HC), lambda b: (0, 0)),
      ],
      out_specs=[
          pl.BlockSpec((R, HGP), lambda b: (b, 0)),
          pl.BlockSpec((R, HC), lambda b: (b, 0)),
      ],
      out_shape=[
          jax.ShapeDtypeStruct((N, HGP), _f32),
          jax.ShapeDtypeStruct((N, HC), _f32),
      ],
  )(x, wgp, w1, b1)


EBLK = 8
EDEPTH = 4
RE_C0 = 136
RE_C1 = 24
PT = 24


def _sc_edge_body(h_hbm, src_hbm, dst_hbm, a_hbm, e_out, sq_out, mx_out,
                  av, sidx, didx, hs0, hs1, hs2, hs3, hd0, hd1, hd2, hd3,
                  ebuf, sqbuf, mbuf, pte, pts, shs0, shs1, shs2, shs3,
                  shd0, shd1, shd2, shd3):
  c = lax.axis_index("c")
  s = lax.axis_index("s")
  wid = c * NS + s
  rowbase = jnp.where(c == 0, s * RE_C0, NS * RE_C0 + s * RE_C1)
  nblocks = jnp.where(c == 0, RE_C0 // EBLK, RE_C1 // EBLK)
  pltpu.sync_copy(a_hbm, av)
  av5 = tuple(av[pl.ds(16 * k, 16)] for k in range(HGP // 16))
  hsb = (hs0, hs1, hs2, hs3)
  hdb = (hd0, hd1, hd2, hd3)
  shs = (shs0, shs1, shs2, shs3)
  shd = (shd0, shd1, shd2, shd3)
  lane15 = _iota16() * PT + 15

  def compute(hsr, hdr, j, m):
    def gbody(g, m):
      for l in range(16):
        r = g * 16 + l
        hs_k = [hsr[r, pl.dslice(16 * k, 16)] for k in range(HGP // 16)]
        hd_k = [hdr[r, pl.dslice(16 * k, 16)] for k in range(HGP // 16)]
        pe = None
        ps = None
        for k in range(HGP // 16):
          d = hs_k[k] - hd_k[k]
          t = jnp.abs(d) * av5[k]
          u = d * d
          pe = t if pe is None else pe + t
          ps = u if ps is None else ps + u
        pte[pl.dslice(l * PT, 16)] = plsc.cumsum(pe)
        pts[pl.dslice(l * PT, 16)] = plsc.cumsum(ps)
      ev = jnp.maximum(plsc.load_gather(pte, [lane15]), 0.0)
      sv = plsc.load_gather(pts, [lane15])
      ebuf[j, pl.dslice(g * 16, 16)] = ev
      sqbuf[j, pl.dslice(g * 16, 16)] = sv
      return jnp.maximum(m, ev)

    return lax.fori_loop(0, K // 16, gbody, m)

  def block(bi, m):
    row0 = rowbase + bi * EBLK
    pltpu.sync_copy(src_hbm.at[pl.ds(row0, EBLK)], sidx)
    pltpu.sync_copy(dst_hbm.at[pl.ds(row0, EBLK)], didx)
    for j in range(EDEPTH - 1):
      pltpu.async_copy(h_hbm.at[sidx.at[j]], hsb[j], shs[j])
      pltpu.async_copy(h_hbm.at[didx.at[j]], hdb[j], shd[j])

    def quad(p, m):
      for b in range(EDEPTH):
        j = EDEPTH * p + b
        ib = (b + EDEPTH - 1) % EDEPTH

        @pl.when(j + EDEPTH - 1 < EBLK)
        def _issue():
          pltpu.async_copy(h_hbm.at[sidx.at[j + EDEPTH - 1]], hsb[ib], shs[ib])
          pltpu.async_copy(h_hbm.at[didx.at[j + EDEPTH - 1]], hdb[ib], shd[ib])

        pltpu.make_async_copy(h_hbm.at[sidx.at[j]], hsb[b], shs[b]).wait()
        pltpu.make_async_copy(h_hbm.at[didx.at[j]], hdb[b], shd[b]).wait()
        m = compute(hsb[b], hdb[b], j, m)
      return m

    m = lax.fori_loop(0, EBLK // EDEPTH, quad, m)
    pltpu.sync_copy(ebuf, e_out.at[pl.ds(row0, EBLK)])
    pltpu.sync_copy(sqbuf, sq_out.at[pl.ds(row0, EBLK)])
    return m

  m = lax.fori_loop(0, nblocks, block, jnp.zeros((16,), _f32))
  mbuf[...] = m
  pltpu.sync_copy(mbuf, mx_out.at[wid])


def _sc_edge(h, src2, dstg2, a_pad):
  return pl.kernel(
      _sc_edge_body,
      out_type=[
          jax.ShapeDtypeStruct((NR, K), _f32),
          jax.ShapeDtypeStruct((NR, K), _f32),
          jax.ShapeDtypeStruct((NW, 16), _f32),
      ],
      mesh=_mesh(),
      scratch_types=[
          pltpu.VMEM((HGP,), _f32),
          pltpu.VMEM((EBLK, K), _i32),
          pltpu.VMEM((EBLK, K), _i32),
          pltpu.VMEM((K, HGP), _f32),
          pltpu.VMEM((K, HGP), _f32),
          pltpu.VMEM((K, HGP), _f32),
          pltpu.VMEM((K, HGP), _f32),
          pltpu.VMEM((K, HGP), _f32),
          pltpu.VMEM((K, HGP), _f32),
          pltpu.VMEM((K, HGP), _f32),
          pltpu.VMEM((K, HGP), _f32),
          pltpu.VMEM((EBLK, K), _f32),
          pltpu.VMEM((EBLK, K), _f32),
          pltpu.VMEM((16,), _f32),
          pltpu.VMEM((16 * PT,), _f32),
          pltpu.VMEM((16 * PT,), _f32),
          pltpu.SemaphoreType.DMA,
          pltpu.SemaphoreType.DMA,
          pltpu.SemaphoreType.DMA,
          pltpu.SemaphoreType.DMA,
          pltpu.SemaphoreType.DMA,
          pltpu.SemaphoreType.DMA,
          pltpu.SemaphoreType.DMA,
          pltpu.SemaphoreType.DMA,
      ],
      compiler_params=_sc_params,
      name="sc_edge_scores",
  )(h, src2, dstg2, a_pad)


def _sc_soft_body(e_hbm, sq_hbm, dst_hbm, mx_hbm, att_out, lp_out,
                  den_sh, den_v, mxv, ebuf, didx, exbuf, sqbuf, attbuf,
                  zbuf, lossbuf, sem_sc):
  c = lax.axis_index("c")
  s = lax.axis_index("s")
  wid = c * NS + s

  pltpu.sync_copy(mx_hbm, mxv)

  def rmax(i, m):
    return jnp.maximum(m, mxv[i, :])

  mvec = lax.fori_loop(0, NW, rmax, jnp.zeros((16,), _f32))
  gmax = lax.reduce_max(mvec, (0,))

  for g in range(8):
    zbuf[pl.ds(g * 16, 16)] = jnp.zeros((16,), _f32)
  for j in range(NA // K):
    pltpu.sync_copy(zbuf, den_sh.at[pl.ds(s * NA + j * K, K)])
  plsc.subcore_barrier()

  def bstep(bi, carry):
    row0 = s * RPS + bi * BLK
    pltpu.sync_copy(e_hbm.at[pl.ds(row0, BLK)], ebuf)
    pltpu.sync_copy(dst_hbm.at[pl.ds(row0, BLK)], didx)

    def jexp(j, carry2):
      for g in range(K // 16):
        exbuf[j, pl.dslice(g * 16, 16)] = jnp.exp(
            ebuf[j, pl.dslice(g * 16, 16)] - gmax)
      return carry2

    lax.fori_loop(0, BLK, jexp, 0)

    def fire(j, carry2):
      pltpu.async_copy(exbuf.at[j], den_sh.at[didx.at[j]], sem_sc, add=True)
      return carry2

    lax.fori_loop(0, BLK, fire, 0)

    def drain(j, carry2):
      pltpu.make_async_copy(exbuf.at[j], den_sh.at[didx.at[j]], sem_sc).wait()
      return carry2

    lax.fori_loop(0, BLK, drain, 0)
    return carry

  lax.fori_loop(0, NBS, bstep, 0)
  plsc.subcore_barrier()

  pltpu.sync_copy(den_sh, den_v)
  rowbase = wid * RPT

  def cstep(bi, carry):
    l1, l2 = carry
    row0 = rowbase + bi * BLK
    pltpu.sync_copy(e_hbm.at[pl.ds(row0, BLK)], ebuf)
    pltpu.sync_copy(dst_hbm.at[pl.ds(row0, BLK)], didx)
    pltpu.sync_copy(sq_hbm.at[pl.ds(row0, BLK)], sqbuf)

    def jstep(j, carry2):
      l1, l2 = carry2
      for g in range(K // 16):
        sl = pl.dslice(g * 16, 16)
        ex = jnp.exp(ebuf[j, sl] - gmax)
        idx = didx[j, sl]
        den = plsc.load_gather(den_v, [idx])
        at = ex / (den + 1e-16)
        attbuf[j, sl] = at
        gidx = (row0 + j) * K + g * 16 + _iota16()
        msk = gidx < E
        l1 = l1 + jnp.where(msk, at * sqbuf[j, sl], 0.0)
        l2 = l2 + jnp.where(msk, at * at, 0.0)
      return l1, l2

    l1, l2 = lax.fori_loop(0, BLK, jstep, (l1, l2))
    pltpu.sync_copy(attbuf, att_out.at[pl.ds(row0, BLK)])
    return l1, l2

  z16 = jnp.zeros((16,), _f32)
  l1, l2 = lax.fori_loop(0, NBG, cstep, (z16, z16))
  lossbuf[pl.ds(0, 16)] = l1
  lossbuf[pl.ds(16, 16)] = l2
  pltpu.sync_copy(lossbuf, lp_out.at[wid])


def _sc_soft(e2, sq2, dsts2, mx):
  return pl.kernel(
      _sc_soft_body,
      out_type=[
          jax.ShapeDtypeStruct((NR, K), _f32),
          jax.ShapeDtypeStruct((NW, 32), _f32),
      ],
      mesh=_mesh(),
      scratch_types=[
          pltpu.VMEM_SHARED((NP_DEN,), _f32),
          pltpu.VMEM((NP_DEN,), _f32),
          pltpu.VMEM((NW, 16), _f32),
          pltpu.VMEM((BLK, K), _f32),
          pltpu.VMEM((BLK, K), _i32),
          pltpu.VMEM((BLK, K), _f32),
          pltpu.VMEM((BLK, K), _f32),
          pltpu.VMEM((BLK, K), _f32),
          pltpu.VMEM((K,), _f32),
          pltpu.VMEM((32,), _f32),
          pltpu.SemaphoreType.DMA,
      ],
      compiler_params=_sc_params,
      name="sc_segment_softmax",
  )(e2, sq2, dsts2, mx)


def _sc_spmm_body(F, RS0, RS1, src_hbm, dst_hbm, att_hbm, y_hbm, out_hbm,
                  acc_sh, sidx, didx, attb, rows0, rows1,
                  sg0, sg1, ss0, ss1):
  c = lax.axis_index("c")
  s = lax.axis_index("s")
  rowsb = (rows0, rows1)
  sg = (sg0, sg1)
  del ss0, ss1

  def run():
    def zrow(r, carry):
      for k in range(F // 16):
        rows0[r, pl.dslice(k * 16, 16)] = jnp.zeros((16,), _f32)
      return carry

    lax.fori_loop(0, K, zrow, 0)
    for j in range(NA // K):
      pltpu.sync_copy(rows0, acc_sh.at[pl.ds(s * NA + j * K, K)])
    plsc.subcore_barrier()
    if RS1 == 0:
      rowbase = s * RS0
      nblocks = RS0 // BLK
    else:
      rowbase = jnp.where(c == 0, s * RS0, NS * RS0 + s * RS1)
      nblocks = jnp.where(c == 0, RS0 // BLK, RS1 // BLK)

    def scale(rref, j):
      def gbody(g, carry):
        atv = attb[j, pl.dslice(g * 16, 16)]
        for l in range(16):
          asp = jnp.full((16,), atv[l], _f32)
          r = g * 16 + l
          for k in range(F // 16):
            sl = pl.dslice(k * 16, 16)
            rref[r, sl] = rref[r, sl] * asp
        return carry

      lax.fori_loop(0, K // 16, gbody, 0)

    def block(bi, carry):
      row0 = rowbase + bi * BLK
      pltpu.sync_copy(src_hbm.at[pl.ds(row0, BLK)], sidx)
      pltpu.sync_copy(dst_hbm.at[pl.ds(row0, BLK)], didx)
      pltpu.sync_copy(att_hbm.at[pl.ds(row0, BLK)], attb)
      pltpu.async_copy(y_hbm.at[sidx.at[0]], rows0, sg0)

      def pair(p, carry2):
        for b in range(2):
          j = 2 * p + b
          nb = 1 - b

          @pl.when(j + 1 < BLK)
          def _issue():
            pltpu.async_copy(y_hbm.at[sidx.at[j + 1]], rowsb[nb], sg[nb])

          pltpu.make_async_copy(y_hbm.at[sidx.at[j]], rowsb[b], sg[b]).wait()
          scale(rowsb[b], j)
          pltpu.sync_copy(rowsb[b], acc_sh.at[didx.at[j]], add=True)
        return carry2

      lax.fori_loop(0, BLK // 2, pair, 0)
      return carry

    lax.fori_loop(0, nblocks, block, 0)
    plsc.subcore_barrier()

    for j in range(NA // K):
      start = s * NA + j * K
      if RS1 == 0:
        pltpu.sync_copy(acc_sh.at[pl.ds(start, K)],
                        out_hbm.at[pl.ds(start, K)])
      else:
        pltpu.sync_copy(acc_sh.at[pl.ds(start, K)],
                        out_hbm.at[pl.ds(c * NP_DEN + start, K)])

  if RS1 == 0:
    @pl.when(c == 0)
    def _run0():
      run()
  else:
    run()


def _sc_spmm(F, RS0, RS1, src2, dsts2, att2, y):
  nout = NP_DEN if RS1 == 0 else NC * NP_DEN
  return pl.kernel(
      functools.partial(_sc_spmm_body, F, RS0, RS1),
      out_type=jax.ShapeDtypeStruct((nout, F), _f32),
      mesh=_mesh(),
      scratch_types=[
          pltpu.VMEM_SHARED((NP_DEN, F), _f32),
          pltpu.VMEM((BLK, K), _i32),
          pltpu.VMEM((BLK, K), _i32),
          pltpu.VMEM((BLK, K), _f32),
          pltpu.VMEM((K, F), _f32),
          pltpu.VMEM((K, F), _f32),
          pltpu.SemaphoreType.DMA,
          pltpu.SemaphoreType.DMA,
          pltpu.SemaphoreType.DMA,
          pltpu.SemaphoreType.DMA,
      ],
      compiler_params=_sc_params,
      name=f"sc_spmm_{F}",
  )(src2, dsts2, att2, y)


def _tc2_body(acc_ref, w2_ref, b2_ref, y2_ref):
  z1 = jnp.maximum(acc_ref[0] + acc_ref[1], 0.0)
  y2_ref[...] = jnp.dot(z1, w2_ref[...], preferred_element_type=_f32) + b2_ref[...]


def _tc2(acc, w2, b2):
  R = 1000
  return pl.pallas_call(
      _tc2_body,
      grid=(N // R,),
      in_specs=[
          pl.BlockSpec((2, R, HC), lambda b: (0, b, 0)),
          pl.BlockSpec((HC, C), lambda b: (0, 0)),
          pl.BlockSpec((1, C), lambda b: (0, 0)),
      ],
      out_specs=pl.BlockSpec((R, C), lambda b: (b, 0)),
      out_shape=jax.ShapeDtypeStruct((N, C), _f32),
  )(acc, w2, b2)


def _tc3_body(acc_ref, lp_ref, z_ref, loss_ref):
  z_ref[...] = acc_ref[0] + acc_ref[1]

  @pl.when(pl.program_id(0) == 0)
  def _():
    lp = lp_ref[...]
    l1 = jnp.sum(lp[:, :16])
    l2 = jnp.sum(lp[:, 16:])
    loss_ref[...] = jnp.reshape(
        (LAMB1 * l1 + LAMB2 * l2) / float(N * N), (1, 1))


def _tc3(acc, lp):
  R = 1000
  return pl.pallas_call(
      _tc3_body,
      grid=(N // R,),
      in_specs=[
          pl.BlockSpec((2, R, C), lambda b: (0, b, 0)),
          pl.BlockSpec((NW, 32), lambda b: (0, 0)),
      ],
      out_specs=[
          pl.BlockSpec((R, C), lambda b: (b, 0)),
          pl.BlockSpec((1, 1), lambda b: (0, 0)),
      ],
      out_shape=[
          jax.ShapeDtypeStruct((N, C), _f32),
          jax.ShapeDtypeStruct((1, 1), _f32),
      ],
  )(acc, lp)


def kernel(x, edge_index, Wg, a, W1, b1, W2, b2):
  src = edge_index[0]
  dst = edge_index[1]
  pad = EP - E
  zpad = jnp.zeros((pad,), _i32)
  src2 = jnp.concatenate([src, zpad]).reshape(NR, K)
  dstg2 = jnp.concatenate([dst, zpad]).reshape(NR, K)
  dsts2 = jnp.concatenate(
      [dst, N + (jnp.arange(pad, dtype=_i32) % (NP_DEN - N))]).reshape(NR, K)
  wgp = jnp.pad(Wg, ((0, 0), (0, HGP - HG)))
  a_pad = jnp.pad(a[:, 0], (0, HGP - HG))

  h, y1 = _tc1(x, wgp, W1, b1.reshape(1, HC))
  e2, sq2, mx = _sc_edge(h, src2, dstg2, a_pad)
  att2, lossparts = _sc_soft(e2, sq2, dsts2, mx)
  acc1 = _sc_spmm(HC, 130, 30, src2, dsts2, att2, y1)
  y2 = _tc2(acc1.reshape(NC, NP_DEN, HC), W2, b2.reshape(1, C))
  acc2 = _sc_spmm(C, 90, 70, src2, dsts2, att2, y2)
  z, loss = _tc3(acc2.reshape(NC, NP_DEN, C), lossparts)
  att = att2.reshape(EP)[:E]
  return z, att, loss[0, 0]
